# baseline (speedup 1.0000x reference)
import jax
import jax.numpy as jnp
from jax.experimental import pallas as pl
from jax.experimental.pallas import tpu as pltpu

_BM = 200


def _gcn_kernel(adj0_ref, adj1_ref, x_ref, w_ref, b_ref, o_ref, s_ref):
    @pl.when(pl.program_id(0) == 0)
    def _():
        s_ref[...] = jnp.dot(x_ref[...], w_ref[...],
                             preferred_element_type=jnp.float32)

    o_ref[:_BM, :] = jnp.dot(adj0_ref[...], s_ref[...],
                             preferred_element_type=jnp.float32) + b_ref[...]
    o_ref[_BM:, :] = jnp.dot(adj1_ref[...], s_ref[...],
                             preferred_element_type=jnp.float32) + b_ref[...]


def kernel(input, adj, W, b):
    n, d_in = input.shape
    d_out = W.shape[1]

    out = pl.pallas_call(
        _gcn_kernel,
        grid=(n // (2 * _BM),),
        in_specs=[
            pl.BlockSpec((_BM, n), lambda i: (2 * i, 0)),
            pl.BlockSpec((_BM, n), lambda i: (2 * i + 1, 0)),
            pl.BlockSpec((n, d_in), lambda i: (0, 0)),
            pl.BlockSpec((d_in, d_out), lambda i: (0, 0)),
            pl.BlockSpec((1, d_out), lambda i: (0, 0)),
        ],
        out_specs=pl.BlockSpec((2 * _BM, d_out), lambda i: (i, 0)),
        out_shape=jax.ShapeDtypeStruct((n, d_out), jnp.float32),
        scratch_shapes=[pltpu.VMEM((n, d_out), jnp.float32)],
        compiler_params=pltpu.CompilerParams(
            dimension_semantics=("arbitrary",),
        ),
    )(adj, adj, input, W, b.reshape(1, d_out))
    return out


# speedup vs baseline: 1.0031x; 1.0031x over previous
import jax
import jax.numpy as jnp
from jax.experimental import pallas as pl
from jax.experimental.pallas import tpu as pltpu

_BM = 400


def _gcn_kernel(adj_ref, x_ref, w_ref, b_ref, o_ref, s_ref):
    @pl.when(pl.program_id(0) == 0)
    def _():
        s_ref[...] = jnp.dot(x_ref[...], w_ref[...],
                             preferred_element_type=jnp.float32)

    o_ref[...] = jnp.dot(adj_ref[...], s_ref[...],
                         preferred_element_type=jnp.float32) + b_ref[...]


def kernel(input, adj, W, b):
    n, d_in = input.shape
    d_out = W.shape[1]

    out = pl.pallas_call(
        _gcn_kernel,
        grid=(n // _BM,),
        in_specs=[
            pl.BlockSpec((_BM, n), lambda i: (i, 0)),
            pl.BlockSpec((n, d_in), lambda i: (0, 0)),
            pl.BlockSpec((d_in, d_out), lambda i: (0, 0)),
            pl.BlockSpec((1, d_out), lambda i: (0, 0)),
        ],
        out_specs=pl.BlockSpec((_BM, d_out), lambda i: (i, 0)),
        out_shape=jax.ShapeDtypeStruct((n, d_out), jnp.float32),
        scratch_shapes=[pltpu.VMEM((n, d_out), jnp.float32)],
        compiler_params=pltpu.CompilerParams(
            dimension_semantics=("arbitrary",),
        ),
    )(adj, input, W, b.reshape(1, d_out))
    return out
